# baseline (speedup 1.0000x reference)
import functools

import numpy as np
import jax
import jax.numpy as jnp
from jax import lax
from jax.experimental import pallas as pl
from jax.experimental.pallas import tpu as pltpu
from jax.experimental.pallas import tpu_sc as plsc

_J = 8
_L = 8
_L1 = 4

_C = 1792
_H = 224
_W = 224
_S = _H * _W


def _ratio_index_lists():
    par1, par2, per1, per2, q1, q2 = [], [], [], [], [], []
    for j1 in range(_J):
        for j2 in range(j1 + 1, _J):
            for l2 in range(_L):
                ci2 = (_L1 * _L * (_J - j1 - 1) + l2 + _L * (j2 - j1 - 1)
                       + _L ** 2 * (j1 * (_J - 1) - j1 * (j1 - 1) // 2))
                ci1 = _L1 + j1 * _L
                if l2 == _L1:
                    par1.append(ci1); par2.append(ci2)
                if l2 == _L1 + _L / 2 or l2 == _L1 - _L / 2:
                    per1.append(ci1); per2.append(ci2)
                if l2 == _L1 + _L // 4 or l2 == _L1 - _L // 4:
                    q1.append(ci1); q2.append(ci2)
    return (np.array(par1, np.int32), np.array(par2, np.int32),
            np.array(per1, np.int32), np.array(per2, np.int32),
            np.array(q1, np.int32), np.array(q2, np.int32))


_P1, _P2, _R1, _R2, _Q1, _Q2 = _ratio_index_lists()
_N_PAR = len(_P1)
_N_PER = len(_R1)
_N_QUART = len(_Q1)

_DEN = np.array(sorted(set(_P1) | set(_R1) | set(_Q1)), np.int32)
_DEN_POS = {int(c): i for i, c in enumerate(_DEN)}
_DPAD = 8 - len(_DEN)

_CHAN = np.concatenate([
    _P2, _R2, _DEN, np.zeros(_DPAD, np.int32),
    _Q2, _DEN, np.zeros(_DPAD, np.int32),
]).astype(np.int32)
assert _CHAN.shape == (128,)

_IDXN = np.concatenate([
    np.arange(56, dtype=np.int32), np.zeros(8, np.int32),
    64 + np.arange(56, dtype=np.int32), np.zeros(8, np.int32)])
_IDXD = np.concatenate([
    np.array([56 + _DEN_POS[int(c)] for c in _P1], np.int32),
    np.array([56 + _DEN_POS[int(c)] for c in _R1], np.int32),
    np.zeros(8, np.int32),
    np.array([120 + _DEN_POS[int(c)] for c in _Q1], np.int32),
    np.zeros(8, np.int32)])


_NCH = 128


def _mean_body(chan_ref, x_ref, o_ref, acc_ref):
    i = pl.program_id(0)
    lanes = lax.broadcasted_iota(jnp.int32, (1, _NCH), 1)
    s = jnp.sum(x_ref[...])

    @pl.when(i == 0)
    def _init():
        acc_ref[...] = jnp.zeros((1, _NCH), jnp.float32)

    acc_ref[...] = jnp.where(lanes == i, s, acc_ref[...])

    @pl.when(i == _NCH - 1)
    def _emit():
        o_ref[...] = acc_ref[...] * (1.0 / _S)


def _gathered_means(x):
    grid_spec = pltpu.PrefetchScalarGridSpec(
        num_scalar_prefetch=1,
        grid=(_NCH,),
        in_specs=[pl.BlockSpec((1, _H, _W), lambda i, chan: (chan[i], 0, 0))],
        out_specs=pl.BlockSpec((1, _NCH), lambda i, chan: (0, 0)),
        scratch_shapes=[pltpu.VMEM((1, _NCH), jnp.float32)],
    )
    return pl.pallas_call(
        _mean_body,
        grid_spec=grid_spec,
        out_shape=jax.ShapeDtypeStruct((1, _NCH), jnp.float32),
    )(jnp.asarray(_CHAN), x)


@functools.lru_cache(maxsize=1)
def _make_ratio_kernel():
    mesh = plsc.VectorSubcoreMesh(core_axis_name="c", subcore_axis_name="s")

    @functools.partial(
        pl.kernel,
        mesh=mesh,
        compiler_params=pltpu.CompilerParams(needs_layout_passes=False),
        out_type=jax.ShapeDtypeStruct((128,), jnp.float32),
        scratch_types=[
            pltpu.VMEM((128,), jnp.float32),
            pltpu.VMEM((128,), jnp.int32),
            pltpu.VMEM((128,), jnp.int32),
            pltpu.VMEM((128,), jnp.float32),
        ],
    )
    def _ratio_kernel(sti_hbm, idxn_hbm, idxd_hbm, out_hbm,
                      sti_v, idxn_v, idxd_v, out_v):
        wid = lax.axis_index("s") * 2 + lax.axis_index("c")

        @pl.when(wid == 0)
        def _work():
            pltpu.sync_copy(sti_hbm, sti_v)
            pltpu.sync_copy(idxn_hbm, idxn_v)
            pltpu.sync_copy(idxd_hbm, idxd_v)
            for i in range(8):
                sl = pl.ds(i * 16, 16)
                num = plsc.load_gather(sti_v, [idxn_v[sl]])
                den = plsc.load_gather(sti_v, [idxd_v[sl]])
                out_v[sl] = num / den
            pltpu.sync_copy(out_v, out_hbm)

    return _ratio_kernel


def kernel(x):
    means = _gathered_means(x).reshape(_NCH)
    ratios = _make_ratio_kernel()(means,
                                  jnp.asarray(_IDXN),
                                  jnp.asarray(_IDXD))
    scat_par = ratios[:_N_PAR]
    scat_per = ratios[28:28 + _N_PER]
    scat_quart = ratios[64:64 + _N_QUART]
    return (scat_par, scat_per, scat_quart)


# speedup vs baseline: 1.1498x; 1.1498x over previous
import functools

import numpy as np
import jax
import jax.numpy as jnp
from jax import lax
from jax.experimental import pallas as pl
from jax.experimental.pallas import tpu as pltpu
from jax.experimental.pallas import tpu_sc as plsc

_J = 8
_L = 8
_L1 = 4

_C = 1792
_H = 224
_W = 224
_S = _H * _W


def _ratio_index_lists():
    par1, par2, per1, per2, q1, q2 = [], [], [], [], [], []
    for j1 in range(_J):
        for j2 in range(j1 + 1, _J):
            for l2 in range(_L):
                ci2 = (_L1 * _L * (_J - j1 - 1) + l2 + _L * (j2 - j1 - 1)
                       + _L ** 2 * (j1 * (_J - 1) - j1 * (j1 - 1) // 2))
                ci1 = _L1 + j1 * _L
                if l2 == _L1:
                    par1.append(ci1); par2.append(ci2)
                if l2 == _L1 + _L / 2 or l2 == _L1 - _L / 2:
                    per1.append(ci1); per2.append(ci2)
                if l2 == _L1 + _L // 4 or l2 == _L1 - _L // 4:
                    q1.append(ci1); q2.append(ci2)
    return (np.array(par1, np.int32), np.array(par2, np.int32),
            np.array(per1, np.int32), np.array(per2, np.int32),
            np.array(q1, np.int32), np.array(q2, np.int32))


_P1, _P2, _R1, _R2, _Q1, _Q2 = _ratio_index_lists()
_N_PAR = len(_P1)
_N_PER = len(_R1)
_N_QUART = len(_Q1)

_DEN = np.array(sorted(set(_P1) | set(_R1) | set(_Q1)), np.int32)
_DEN_POS = {int(c): i for i, c in enumerate(_DEN)}
_DPAD = 8 - len(_DEN)

_CHAN = np.concatenate([
    _P2, _R2, _DEN, np.zeros(_DPAD, np.int32),
    _Q2, _DEN, np.zeros(_DPAD, np.int32),
]).astype(np.int32)
assert _CHAN.shape == (128,)

_IDXN = np.concatenate([
    np.arange(56, dtype=np.int32), np.zeros(8, np.int32),
    64 + np.arange(56, dtype=np.int32), np.zeros(8, np.int32)])
_IDXD = np.concatenate([
    np.array([56 + _DEN_POS[int(c)] for c in _P1], np.int32),
    np.array([56 + _DEN_POS[int(c)] for c in _R1], np.int32),
    np.zeros(8, np.int32),
    np.array([120 + _DEN_POS[int(c)] for c in _Q1], np.int32),
    np.zeros(8, np.int32)])


_NCH = 128
_KOP = 8
_NST = _NCH // _KOP


def _mean_body(chan_ref, *refs):
    x_refs = refs[:_KOP]
    o_ref = refs[_KOP]
    acc_ref = refs[_KOP + 1]
    i = pl.program_id(0)
    lanes = lax.broadcasted_iota(jnp.int32, (1, _NCH), 1)

    @pl.when(i == 0)
    def _init():
        acc_ref[...] = jnp.zeros((1, _NCH), jnp.float32)

    acc = acc_ref[...]
    for q in range(_KOP):
        s = jnp.sum(x_refs[q][...])
        acc = jnp.where(lanes == i * _KOP + q, s, acc)
    acc_ref[...] = acc

    @pl.when(i == _NST - 1)
    def _emit():
        o_ref[...] = acc_ref[...] * (1.0 / _S)


def _gathered_means(x):
    def _imap(q):
        return lambda i, chan: (chan[i * _KOP + q], 0, 0)

    grid_spec = pltpu.PrefetchScalarGridSpec(
        num_scalar_prefetch=1,
        grid=(_NST,),
        in_specs=[pl.BlockSpec((1, _H, _W), _imap(q)) for q in range(_KOP)],
        out_specs=pl.BlockSpec((1, _NCH), lambda i, chan: (0, 0)),
        scratch_shapes=[pltpu.VMEM((1, _NCH), jnp.float32)],
    )
    return pl.pallas_call(
        _mean_body,
        grid_spec=grid_spec,
        out_shape=jax.ShapeDtypeStruct((1, _NCH), jnp.float32),
    )(jnp.asarray(_CHAN), *([x] * _KOP))


@functools.lru_cache(maxsize=1)
def _make_ratio_kernel():
    mesh = plsc.VectorSubcoreMesh(core_axis_name="c", subcore_axis_name="s")

    @functools.partial(
        pl.kernel,
        mesh=mesh,
        compiler_params=pltpu.CompilerParams(needs_layout_passes=False),
        out_type=jax.ShapeDtypeStruct((128,), jnp.float32),
        scratch_types=[
            pltpu.VMEM((128,), jnp.float32),
            pltpu.VMEM((128,), jnp.int32),
            pltpu.VMEM((128,), jnp.int32),
            pltpu.VMEM((128,), jnp.float32),
        ],
    )
    def _ratio_kernel(sti_hbm, idxn_hbm, idxd_hbm, out_hbm,
                      sti_v, idxn_v, idxd_v, out_v):
        wid = lax.axis_index("s") * 2 + lax.axis_index("c")

        @pl.when(wid == 0)
        def _work():
            pltpu.sync_copy(sti_hbm, sti_v)
            pltpu.sync_copy(idxn_hbm, idxn_v)
            pltpu.sync_copy(idxd_hbm, idxd_v)
            for i in range(8):
                sl = pl.ds(i * 16, 16)
                num = plsc.load_gather(sti_v, [idxn_v[sl]])
                den = plsc.load_gather(sti_v, [idxd_v[sl]])
                out_v[sl] = num / den
            pltpu.sync_copy(out_v, out_hbm)

    return _ratio_kernel


def kernel(x):
    means = _gathered_means(x).reshape(_NCH)
    ratios = _make_ratio_kernel()(means,
                                  jnp.asarray(_IDXN),
                                  jnp.asarray(_IDXD))
    scat_par = ratios[:_N_PAR]
    scat_per = ratios[28:28 + _N_PER]
    scat_quart = ratios[64:64 + _N_QUART]
    return (scat_par, scat_per, scat_quart)
